# baseline (speedup 1.0000x reference)
import functools

import jax
import jax.numpy as jnp
from jax import lax
from jax.experimental import pallas as pl
from jax.experimental.pallas import tpu as pltpu
from jax.experimental.pallas import tpu_sc as plsc

N = 10000
D = 128
H = 128
E = 320000
NC = 2
NS = 16
NW = NC * NS
EPW = E // NW
EP = 327680
EPWP = EP // NW
NSUP = 40
PADDST = N + 16
NPAD = 10240
RPT = NPAD // NS
ZR = 64
RB = 10
BR = N // RB
NEG_SLOPE = 0.2


def _attn_scalars(h, a_s, a_d):
    as_ = jnp.sum(h * a_s[None, :], axis=1, keepdims=True)
    ad_ = jnp.sum(h * a_d[None, :], axis=1, keepdims=True)
    e = as_ + ad_
    ps = jnp.exp(jnp.where(e >= 0, e, NEG_SLOPE * e))
    br = h.shape[0]
    return jnp.concatenate([as_, ad_, ps, jnp.zeros((br, 5), jnp.float32)], axis=1)


def _tc1_body(x_ref, w_ref, as_ref, ad_ref, h_ref, sc_ref):
    h = jnp.dot(x_ref[...], w_ref[...], preferred_element_type=jnp.float32)
    h_ref[...] = h
    sc_ref[...] = _attn_scalars(h, as_ref[...], ad_ref[...])


def _combine(a0, a1, den, h_prev, sc, b):
    ps = sc[:, 2:3]
    num = a0 + a1 + ps * h_prev
    dd = den + ps + 1e-16
    return jnp.maximum(num / dd + b[None, :], 0.0)


def _tc2_body(a0_ref, a1_ref, den_ref, h_ref, sc_ref, b_ref, w_ref, as_ref,
              ad_ref, h2_ref, sc2_ref):
    x2 = _combine(a0_ref[...], a1_ref[...], den_ref[...], h_ref[...],
                  sc_ref[...], b_ref[...])
    h2 = jnp.dot(x2, w_ref[...], preferred_element_type=jnp.float32)
    h2_ref[...] = h2
    sc2_ref[...] = _attn_scalars(h2, as_ref[...], ad_ref[...])


def _tc3_body(a0_ref, a1_ref, den_ref, h_ref, sc_ref, b_ref, fu_ref, fv_ref,
              fb_ref, uv_ref):
    x3 = _combine(a0_ref[...], a1_ref[...], den_ref[...], h_ref[...],
                  sc_ref[...], b_ref[...])
    u = jnp.sum(x3 * fu_ref[...][None, :], axis=1, keepdims=True) + fb_ref[0]
    v = jnp.sum(x3 * fv_ref[...][None, :], axis=1, keepdims=True)
    br = x3.shape[0]
    uv_ref[...] = jnp.concatenate([u, v, jnp.zeros((br, 6), jnp.float32)], axis=1)


def _densum_body(dp_ref, out_ref):
    out_ref[...] = jnp.sum(dp_ref[...], axis=0)


_row_spec = pl.BlockSpec((BR, 128), lambda i: (i, 0))
_sc_spec = pl.BlockSpec((BR, 8), lambda i: (i, 0))
_den_spec = pl.BlockSpec((BR, 1), lambda i: (i, 0))
_w_spec = pl.BlockSpec((128, 128), lambda i: (0, 0))
_v_spec = pl.BlockSpec((128,), lambda i: (0,))


def _tc1(x, W, a_s, a_d):
    return pl.pallas_call(
        _tc1_body,
        grid=(RB,),
        in_specs=[_row_spec, _w_spec, _v_spec, _v_spec],
        out_specs=[_row_spec, _sc_spec],
        out_shape=[
            jax.ShapeDtypeStruct((N, 128), jnp.float32),
            jax.ShapeDtypeStruct((N, 8), jnp.float32),
        ],
    )(x, W, a_s, a_d)


def _densum(dp):
    return pl.pallas_call(
        _densum_body,
        grid=(10,),
        in_specs=[pl.BlockSpec((NW, 1024), lambda i: (0, i))],
        out_specs=[pl.BlockSpec((1024,), lambda i: (i,))],
        out_shape=[jax.ShapeDtypeStruct((NPAD,), jnp.float32)],
    )(dp)[0]


def _tc2(a0, a1, den, h, sc, b, W, a_s, a_d):
    return pl.pallas_call(
        _tc2_body,
        grid=(RB,),
        in_specs=[_row_spec, _row_spec, _den_spec, _row_spec, _sc_spec,
                  _v_spec, _w_spec, _v_spec, _v_spec],
        out_specs=[_row_spec, _sc_spec],
        out_shape=[
            jax.ShapeDtypeStruct((N, 128), jnp.float32),
            jax.ShapeDtypeStruct((N, 8), jnp.float32),
        ],
    )(a0, a1, den, h, sc, b, W, a_s, a_d)


def _tc3(a0, a1, den, h, sc, b, fu, fv, fb):
    return pl.pallas_call(
        _tc3_body,
        grid=(RB,),
        in_specs=[_row_spec, _row_spec, _den_spec, _row_spec, _sc_spec,
                  _v_spec, _v_spec, _v_spec,
                  pl.BlockSpec(memory_space=pltpu.MemorySpace.SMEM)],
        out_specs=[_sc_spec],
        out_shape=[jax.ShapeDtypeStruct((N, 8), jnp.float32)],
    )(a0, a1, den, h, sc, b, fu, fv, fb)[0]


_sc_mesh = plsc.VectorSubcoreMesh(core_axis_name="c", subcore_axis_name="s")


@functools.partial(
    pl.kernel,
    out_type=(
        jax.ShapeDtypeStruct((NC * NPAD, 128), jnp.float32),
        jax.ShapeDtypeStruct((NW * NPAD,), jnp.float32),
    ),
    mesh=_sc_mesh,
    compiler_params=pltpu.CompilerParams(needs_layout_passes=False),
    scratch_types=[
        pltpu.VMEM((128,), jnp.int32),
        pltpu.VMEM((128,), jnp.int32),
        pltpu.VMEM((128,), jnp.int32),
        pltpu.VMEM((128,), jnp.int32),
        pltpu.VMEM((128,), jnp.float32),
        pltpu.VMEM((128,), jnp.float32),
        pltpu.VMEM((128,), jnp.float32),
        pltpu.VMEM((128,), jnp.float32),
        pltpu.VMEM((128,), jnp.float32),
        pltpu.VMEM((128, 128), jnp.float32),
        pltpu.VMEM((128, 128), jnp.float32),
        pltpu.VMEM((NPAD,), jnp.float32),
        pltpu.VMEM_SHARED((NPAD, 128), jnp.float32),
        pltpu.SemaphoreType.DMA,
        pltpu.SemaphoreType.DMA,
        pltpu.SemaphoreType.DMA,
        pltpu.SemaphoreType.DMA,
        pltpu.SemaphoreType.DMA,
        pltpu.SemaphoreType.DMA,
    ],
)
def _edge_pass(h_hbm, as_hbm, ad_hbm, src_hbm, dst_hbm, acc_hbm, den_hbm,
               src_a, dst_a, src_b, dst_b, asg_a, adg_a, asg_b, adg_b, p_v,
               rows_a, rows_b, den_v, acc_sh,
               sem_r0, sem_r1, sem_a0, sem_a1, sem_d0, sem_d1):
    c = lax.axis_index("c")
    s = lax.axis_index("s")
    wid = s * NC + c
    NS0 = 104
    NS1 = (2 * EPWP // 128) - NS0
    ns = jnp.where(c == 0, NS0, NS1)
    sub0 = jnp.where(c == 0, s * NS0, NS * NS0 + s * NS1)

    zeros16 = jnp.zeros((16,), jnp.float32)

    def zero_row(r, carry):
        for j in range(128 // 16):
            rows_a[r, pl.ds(j * 16, 16)] = zeros16
        return carry

    lax.fori_loop(0, 128, zero_row, 0)
    for k in range(RPT // 128):
        pltpu.sync_copy(rows_a, acc_sh.at[pl.ds(s * RPT + k * 128, 128)])

    def zero_den(r, carry):
        den_v[pl.ds(r * 16, 16)] = zeros16
        return carry

    lax.fori_loop(0, NPAD // 16, zero_den, 0)
    plsc.subcore_barrier()

    lane_iota = lax.iota(jnp.int32, 16)
    slots = (
        (src_a, dst_a, asg_a, adg_a, rows_a, sem_r0, sem_a0, sem_d0),
        (src_b, dst_b, asg_b, adg_b, rows_b, sem_r1, sem_a1, sem_d1),
    )

    def stage_and_fire(slot, sub):
        src_s, dst_s, asg_s, adg_s, rows_s, sem_r, sem_a, sem_d = slot
        base = (sub0 + sub) * 128
        pltpu.sync_copy(src_hbm.at[pl.ds(base, 128)], src_s)
        pltpu.sync_copy(dst_hbm.at[pl.ds(base, 128)], dst_s)
        pltpu.async_copy(h_hbm.at[src_s], rows_s, sem_r)
        pltpu.async_copy(as_hbm.at[src_s], asg_s, sem_a)
        pltpu.async_copy(ad_hbm.at[dst_s], adg_s, sem_d)

    def consume(slot):
        src_s, dst_s, asg_s, adg_s, rows_s, sem_r, sem_a, sem_d = slot
        pltpu.make_async_copy(as_hbm.at[pl.ds(0, 128)], asg_s, sem_a).wait()
        pltpu.make_async_copy(ad_hbm.at[pl.ds(0, 128)], adg_s, sem_d).wait()
        pltpu.make_async_copy(h_hbm.at[pl.ds(0, 128)], rows_s, sem_r).wait()

        def group_p(g, carry2):
            e16 = asg_s[pl.ds(g * 16, 16)] + adg_s[pl.ds(g * 16, 16)]
            e16 = jnp.where(e16 >= 0, e16, NEG_SLOPE * e16)
            p16 = jnp.exp(e16)
            p_v[pl.ds(g * 16, 16)] = p16
            d16 = dst_s[pl.ds(g * 16, 16)]
            for l in range(16):
                plsc.addupdate_scatter(den_v, [d16], p16,
                                       mask=lane_iota == l)
            return carry2

        lax.fori_loop(0, 8, group_p, 0)

        def group_scale(g, carry2):
            p16 = p_v[pl.ds(g * 16, 16)]
            for l in range(16):
                pb = jnp.full((16,), p16[l], jnp.float32)
                r = g * 16 + l
                for jj in range(128 // 16):
                    rows_s[r, pl.ds(jj * 16, 16)] = (
                        rows_s[r, pl.ds(jj * 16, 16)] * pb)
            return carry2

        lax.fori_loop(0, 8, group_scale, 0)
        pltpu.sync_copy(rows_s, acc_sh.at[dst_s], add=True)

    stage_and_fire(slots[0], 0)
    stage_and_fire(slots[1], 1)

    def pair_body(u, carry):
        consume(slots[0])
        stage_and_fire(slots[0], jnp.minimum(2 * u + 2, ns - 2))
        consume(slots[1])
        stage_and_fire(slots[1], jnp.minimum(2 * u + 3, ns - 1))
        return carry

    lax.fori_loop(0, ns // 2 - 1, pair_body, 0)
    consume(slots[0])
    consume(slots[1])

    pltpu.sync_copy(den_v, den_hbm.at[pl.ds(wid * NPAD, NPAD)])
    plsc.subcore_barrier()
    r0 = s * RPT
    pltpu.sync_copy(acc_sh.at[pl.ds(r0, RPT)],
                    acc_hbm.at[pl.ds(c * NPAD + r0, RPT)])


@functools.partial(
    pl.kernel,
    out_type=jax.ShapeDtypeStruct((EP,), jnp.float32),
    mesh=_sc_mesh,
    compiler_params=pltpu.CompilerParams(needs_layout_passes=False),
    scratch_types=[
        pltpu.VMEM((NPAD,), jnp.float32),
        pltpu.VMEM((NPAD,), jnp.float32),
        pltpu.VMEM((256,), jnp.int32),
        pltpu.VMEM((256,), jnp.int32),
        pltpu.VMEM((256,), jnp.float32),
    ],
)
def _readout(u_hbm, v_hbm, src_hbm, dst_hbm, out_hbm,
             u_v, v_v, src_v, dst_v, o_v):
    c = lax.axis_index("c")
    s = lax.axis_index("s")
    wid = s * NC + c
    pltpu.sync_copy(u_hbm, u_v)
    pltpu.sync_copy(v_hbm, v_v)

    def super_body(t, carry):
        base = wid * EPWP + t * 256
        pltpu.sync_copy(src_hbm.at[pl.ds(base, 256)], src_v)
        pltpu.sync_copy(dst_hbm.at[pl.ds(base, 256)], dst_v)

        def group_o(g, carry2):
            i16 = src_v[pl.ds(g * 16, 16)]
            d16 = dst_v[pl.ds(g * 16, 16)]
            o_v[pl.ds(g * 16, 16)] = (plsc.load_gather(u_v, [i16])
                                      + plsc.load_gather(v_v, [d16]))
            return carry2

        lax.fori_loop(0, 16, group_o, 0)
        pltpu.sync_copy(o_v, out_hbm.at[pl.ds(base, 256)])
        return carry

    lax.fori_loop(0, NSUP, super_body, 0)


def kernel(x, edge_index, W1, a1_src, a1_dst, b1, W2, a2_src, a2_dst, b2,
           fc_W, fc_b):
    src = edge_index[0]
    dst = edge_index[1]
    pad = EP - E
    srcp = jnp.concatenate([src, jnp.zeros((pad,), src.dtype)])
    dstp = jnp.concatenate([dst, jnp.full((pad,), PADDST, dst.dtype)])
    zpad = jnp.zeros((NPAD - N,), jnp.float32)

    h1, sc1 = _tc1(x, W1, a1_src, a1_dst)
    as1 = jnp.concatenate([sc1[:, 0], zpad])
    ad1 = jnp.concatenate([sc1[:, 1], zpad])
    acc1, dp1 = _edge_pass(h1, as1, ad1, srcp, dstp)
    den1 = _densum(dp1.reshape(NW, NPAD))[:N].reshape(N, 1)
    h2, sc2 = _tc2(acc1[:N], acc1[NPAD:NPAD + N], den1, h1, sc1, b1, W2,
                   a2_src, a2_dst)
    as2 = jnp.concatenate([sc2[:, 0], zpad])
    ad2 = jnp.concatenate([sc2[:, 1], zpad])
    acc2, dp2 = _edge_pass(h2, as2, ad2, srcp, dstp)
    den2 = _densum(dp2.reshape(NW, NPAD))[:N].reshape(N, 1)
    uv = _tc3(acc2[:N], acc2[NPAD:NPAD + N], den2, h2, sc2, b2,
              fc_W[:128, 0], fc_W[128:, 0], fc_b)
    up = jnp.concatenate([uv[:, 0], zpad])
    vp = jnp.concatenate([uv[:, 1], zpad])
    out = _readout(up, vp, srcp, dstp)
    return out[:E, None]


# speedup vs baseline: 1.0121x; 1.0121x over previous
import functools

import jax
import jax.numpy as jnp
from jax import lax
from jax.experimental import pallas as pl
from jax.experimental.pallas import tpu as pltpu
from jax.experimental.pallas import tpu_sc as plsc

N = 10000
D = 128
H = 128
E = 320000
NC = 2
NS = 16
NW = NC * NS
EPW = E // NW
EP = 327680
EPWP = EP // NW
NSUP = 40
PADDST = N + 16
NPAD = 10240
RPT = NPAD // NS
ZR = 64
RB = 10
BR = N // RB
NEG_SLOPE = 0.2


def _attn_scalars(h, a_s, a_d):
    as_ = jnp.sum(h * a_s[None, :], axis=1, keepdims=True)
    ad_ = jnp.sum(h * a_d[None, :], axis=1, keepdims=True)
    e = as_ + ad_
    ps = jnp.exp(jnp.where(e >= 0, e, NEG_SLOPE * e))
    br = h.shape[0]
    return jnp.concatenate([as_, ad_, ps, jnp.zeros((br, 5), jnp.float32)], axis=1)


def _tc1_body(x_ref, w_ref, as_ref, ad_ref, h_ref, sc_ref):
    h = jnp.dot(x_ref[...], w_ref[...], preferred_element_type=jnp.float32)
    h_ref[...] = h
    sc_ref[...] = _attn_scalars(h, as_ref[...], ad_ref[...])


def _combine(a0, a1, den, h_prev, sc, b):
    ps = sc[:, 2:3]
    num = a0 + a1 + ps * h_prev
    dd = den + ps + 1e-16
    return jnp.maximum(num / dd + b[None, :], 0.0)


def _tc2_body(a0_ref, a1_ref, den_ref, h_ref, sc_ref, b_ref, w_ref, as_ref,
              ad_ref, h2_ref, sc2_ref):
    x2 = _combine(a0_ref[...], a1_ref[...], den_ref[...], h_ref[...],
                  sc_ref[...], b_ref[...])
    h2 = jnp.dot(x2, w_ref[...], preferred_element_type=jnp.float32)
    h2_ref[...] = h2
    sc2_ref[...] = _attn_scalars(h2, as_ref[...], ad_ref[...])


def _tc3_body(a0_ref, a1_ref, den_ref, h_ref, sc_ref, b_ref, fu_ref, fv_ref,
              fb_ref, uv_ref):
    x3 = _combine(a0_ref[...], a1_ref[...], den_ref[...], h_ref[...],
                  sc_ref[...], b_ref[...])
    u = jnp.sum(x3 * fu_ref[...][None, :], axis=1, keepdims=True) + fb_ref[0]
    v = jnp.sum(x3 * fv_ref[...][None, :], axis=1, keepdims=True)
    br = x3.shape[0]
    uv_ref[...] = jnp.concatenate([u, v, jnp.zeros((br, 6), jnp.float32)], axis=1)


def _densum_body(dp_ref, out_ref):
    out_ref[...] = jnp.sum(dp_ref[...], axis=0)


_row_spec = pl.BlockSpec((BR, 128), lambda i: (i, 0))
_sc_spec = pl.BlockSpec((BR, 8), lambda i: (i, 0))
_den_spec = pl.BlockSpec((BR, 1), lambda i: (i, 0))
_w_spec = pl.BlockSpec((128, 128), lambda i: (0, 0))
_v_spec = pl.BlockSpec((128,), lambda i: (0,))


def _tc1(x, W, a_s, a_d):
    return pl.pallas_call(
        _tc1_body,
        grid=(RB,),
        in_specs=[_row_spec, _w_spec, _v_spec, _v_spec],
        out_specs=[_row_spec, _sc_spec],
        out_shape=[
            jax.ShapeDtypeStruct((N, 128), jnp.float32),
            jax.ShapeDtypeStruct((N, 8), jnp.float32),
        ],
    )(x, W, a_s, a_d)


def _densum(dp):
    return pl.pallas_call(
        _densum_body,
        grid=(10,),
        in_specs=[pl.BlockSpec((NW, 1024), lambda i: (0, i))],
        out_specs=[pl.BlockSpec((1024,), lambda i: (i,))],
        out_shape=[jax.ShapeDtypeStruct((NPAD,), jnp.float32)],
    )(dp)[0]


def _tc2(a0, a1, den, h, sc, b, W, a_s, a_d):
    return pl.pallas_call(
        _tc2_body,
        grid=(RB,),
        in_specs=[_row_spec, _row_spec, _den_spec, _row_spec, _sc_spec,
                  _v_spec, _w_spec, _v_spec, _v_spec],
        out_specs=[_row_spec, _sc_spec],
        out_shape=[
            jax.ShapeDtypeStruct((N, 128), jnp.float32),
            jax.ShapeDtypeStruct((N, 8), jnp.float32),
        ],
    )(a0, a1, den, h, sc, b, W, a_s, a_d)


def _tc3(a0, a1, den, h, sc, b, fu, fv, fb):
    return pl.pallas_call(
        _tc3_body,
        grid=(RB,),
        in_specs=[_row_spec, _row_spec, _den_spec, _row_spec, _sc_spec,
                  _v_spec, _v_spec, _v_spec,
                  pl.BlockSpec(memory_space=pltpu.MemorySpace.SMEM)],
        out_specs=[_sc_spec],
        out_shape=[jax.ShapeDtypeStruct((N, 8), jnp.float32)],
    )(a0, a1, den, h, sc, b, fu, fv, fb)[0]


_sc_mesh = plsc.VectorSubcoreMesh(core_axis_name="c", subcore_axis_name="s")


@functools.partial(
    pl.kernel,
    out_type=(
        jax.ShapeDtypeStruct((NC * NPAD, 128), jnp.float32),
        jax.ShapeDtypeStruct((NW * NPAD,), jnp.float32),
    ),
    mesh=_sc_mesh,
    compiler_params=pltpu.CompilerParams(needs_layout_passes=False),
    scratch_types=[
        pltpu.VMEM((128,), jnp.int32),
        pltpu.VMEM((128,), jnp.int32),
        pltpu.VMEM((128,), jnp.int32),
        pltpu.VMEM((128,), jnp.int32),
        pltpu.VMEM((128,), jnp.float32),
        pltpu.VMEM((128,), jnp.float32),
        pltpu.VMEM((128,), jnp.float32),
        pltpu.VMEM((128,), jnp.float32),
        pltpu.VMEM((128,), jnp.float32),
        pltpu.VMEM((128, 128), jnp.float32),
        pltpu.VMEM((128, 128), jnp.float32),
        pltpu.VMEM((NPAD,), jnp.float32),
        pltpu.VMEM_SHARED((NPAD, 128), jnp.float32),
        pltpu.SemaphoreType.DMA,
        pltpu.SemaphoreType.DMA,
        pltpu.SemaphoreType.DMA,
        pltpu.SemaphoreType.DMA,
        pltpu.SemaphoreType.DMA,
        pltpu.SemaphoreType.DMA,
    ],
)
def _edge_pass(h_hbm, as_hbm, ad_hbm, src_hbm, dst_hbm, acc_hbm, den_hbm,
               src_a, dst_a, src_b, dst_b, asg_a, adg_a, asg_b, adg_b, p_v,
               rows_a, rows_b, den_v, acc_sh,
               sem_r0, sem_r1, sem_a0, sem_a1, sem_d0, sem_d1):
    c = lax.axis_index("c")
    s = lax.axis_index("s")
    wid = s * NC + c
    NS0 = 114
    NS1 = (2 * EPWP // 128) - NS0
    ns = jnp.where(c == 0, NS0, NS1)
    sub0 = jnp.where(c == 0, s * NS0, NS * NS0 + s * NS1)

    zeros16 = jnp.zeros((16,), jnp.float32)

    def zero_row(r, carry):
        for j in range(128 // 16):
            rows_a[r, pl.ds(j * 16, 16)] = zeros16
        return carry

    lax.fori_loop(0, 128, zero_row, 0)
    for k in range(RPT // 128):
        pltpu.sync_copy(rows_a, acc_sh.at[pl.ds(s * RPT + k * 128, 128)])

    def zero_den(r, carry):
        den_v[pl.ds(r * 16, 16)] = zeros16
        return carry

    lax.fori_loop(0, NPAD // 16, zero_den, 0)
    plsc.subcore_barrier()

    lane_iota = lax.iota(jnp.int32, 16)
    slots = (
        (src_a, dst_a, asg_a, adg_a, rows_a, sem_r0, sem_a0, sem_d0),
        (src_b, dst_b, asg_b, adg_b, rows_b, sem_r1, sem_a1, sem_d1),
    )

    def stage_and_fire(slot, sub):
        src_s, dst_s, asg_s, adg_s, rows_s, sem_r, sem_a, sem_d = slot
        base = (sub0 + sub) * 128
        pltpu.sync_copy(src_hbm.at[pl.ds(base, 128)], src_s)
        pltpu.sync_copy(dst_hbm.at[pl.ds(base, 128)], dst_s)
        pltpu.async_copy(h_hbm.at[src_s.at[pl.ds(0, 64)]],
                         rows_s.at[pl.ds(0, 64)], sem_r)
        pltpu.async_copy(h_hbm.at[src_s.at[pl.ds(64, 64)]],
                         rows_s.at[pl.ds(64, 64)], sem_r)
        pltpu.async_copy(as_hbm.at[src_s], asg_s, sem_a)
        pltpu.async_copy(ad_hbm.at[dst_s], adg_s, sem_d)

    def consume(slot):
        src_s, dst_s, asg_s, adg_s, rows_s, sem_r, sem_a, sem_d = slot
        pltpu.make_async_copy(as_hbm.at[pl.ds(0, 128)], asg_s, sem_a).wait()
        pltpu.make_async_copy(ad_hbm.at[pl.ds(0, 128)], adg_s, sem_d).wait()
        pltpu.make_async_copy(h_hbm.at[pl.ds(0, 64)],
                              rows_s.at[pl.ds(0, 64)], sem_r).wait()
        pltpu.make_async_copy(h_hbm.at[pl.ds(0, 64)],
                              rows_s.at[pl.ds(64, 64)], sem_r).wait()

        def group_p(g, carry2):
            e16 = asg_s[pl.ds(g * 16, 16)] + adg_s[pl.ds(g * 16, 16)]
            e16 = jnp.where(e16 >= 0, e16, NEG_SLOPE * e16)
            p16 = jnp.exp(e16)
            p_v[pl.ds(g * 16, 16)] = p16
            d16 = dst_s[pl.ds(g * 16, 16)]
            for l in range(16):
                plsc.addupdate_scatter(den_v, [d16], p16,
                                       mask=lane_iota == l)
            return carry2

        lax.fori_loop(0, 8, group_p, 0)

        def group_scale(g, carry2):
            p16 = p_v[pl.ds(g * 16, 16)]
            for l in range(16):
                pb = jnp.full((16,), p16[l], jnp.float32)
                r = g * 16 + l
                for jj in range(128 // 16):
                    rows_s[r, pl.ds(jj * 16, 16)] = (
                        rows_s[r, pl.ds(jj * 16, 16)] * pb)
            return carry2

        lax.fori_loop(0, 8, group_scale, 0)
        pltpu.sync_copy(rows_s, acc_sh.at[dst_s], add=True)

    stage_and_fire(slots[0], 0)
    stage_and_fire(slots[1], 1)

    def pair_body(u, carry):
        consume(slots[0])
        stage_and_fire(slots[0], jnp.minimum(2 * u + 2, ns - 2))
        consume(slots[1])
        stage_and_fire(slots[1], jnp.minimum(2 * u + 3, ns - 1))
        return carry

    lax.fori_loop(0, ns // 2 - 1, pair_body, 0)
    consume(slots[0])
    consume(slots[1])

    pltpu.sync_copy(den_v, den_hbm.at[pl.ds(wid * NPAD, NPAD)])
    plsc.subcore_barrier()
    r0 = s * RPT
    pltpu.sync_copy(acc_sh.at[pl.ds(r0, RPT)],
                    acc_hbm.at[pl.ds(c * NPAD + r0, RPT)])


@functools.partial(
    pl.kernel,
    out_type=jax.ShapeDtypeStruct((EP,), jnp.float32),
    mesh=_sc_mesh,
    compiler_params=pltpu.CompilerParams(needs_layout_passes=False),
    scratch_types=[
        pltpu.VMEM((NPAD,), jnp.float32),
        pltpu.VMEM((NPAD,), jnp.float32),
        pltpu.VMEM((256,), jnp.int32),
        pltpu.VMEM((256,), jnp.int32),
        pltpu.VMEM((256,), jnp.float32),
    ],
)
def _readout(u_hbm, v_hbm, src_hbm, dst_hbm, out_hbm,
             u_v, v_v, src_v, dst_v, o_v):
    c = lax.axis_index("c")
    s = lax.axis_index("s")
    wid = s * NC + c
    pltpu.sync_copy(u_hbm, u_v)
    pltpu.sync_copy(v_hbm, v_v)

    def super_body(t, carry):
        base = wid * EPWP + t * 256
        pltpu.sync_copy(src_hbm.at[pl.ds(base, 256)], src_v)
        pltpu.sync_copy(dst_hbm.at[pl.ds(base, 256)], dst_v)

        def group_o(g, carry2):
            i16 = src_v[pl.ds(g * 16, 16)]
            d16 = dst_v[pl.ds(g * 16, 16)]
            o_v[pl.ds(g * 16, 16)] = (plsc.load_gather(u_v, [i16])
                                      + plsc.load_gather(v_v, [d16]))
            return carry2

        lax.fori_loop(0, 16, group_o, 0)
        pltpu.sync_copy(o_v, out_hbm.at[pl.ds(base, 256)])
        return carry

    lax.fori_loop(0, NSUP, super_body, 0)


def kernel(x, edge_index, W1, a1_src, a1_dst, b1, W2, a2_src, a2_dst, b2,
           fc_W, fc_b):
    src = edge_index[0]
    dst = edge_index[1]
    pad = EP - E
    srcp = jnp.concatenate([src, jnp.zeros((pad,), src.dtype)])
    dstp = jnp.concatenate([dst, jnp.full((pad,), PADDST, dst.dtype)])
    zpad = jnp.zeros((NPAD - N,), jnp.float32)

    h1, sc1 = _tc1(x, W1, a1_src, a1_dst)
    as1 = jnp.concatenate([sc1[:, 0], zpad])
    ad1 = jnp.concatenate([sc1[:, 1], zpad])
    acc1, dp1 = _edge_pass(h1, as1, ad1, srcp, dstp)
    den1 = _densum(dp1.reshape(NW, NPAD))[:N].reshape(N, 1)
    h2, sc2 = _tc2(acc1[:N], acc1[NPAD:NPAD + N], den1, h1, sc1, b1, W2,
                   a2_src, a2_dst)
    as2 = jnp.concatenate([sc2[:, 0], zpad])
    ad2 = jnp.concatenate([sc2[:, 1], zpad])
    acc2, dp2 = _edge_pass(h2, as2, ad2, srcp, dstp)
    den2 = _densum(dp2.reshape(NW, NPAD))[:N].reshape(N, 1)
    uv = _tc3(acc2[:N], acc2[NPAD:NPAD + N], den2, h2, sc2, b2,
              fc_W[:128, 0], fc_W[128:, 0], fc_b)
    up = jnp.concatenate([uv[:, 0], zpad])
    vp = jnp.concatenate([uv[:, 1], zpad])
    out = _readout(up, vp, srcp, dstp)
    return out[:E, None]
